# baseline (speedup 1.0000x reference)
import jax
import jax.numpy as jnp
from jax.experimental import pallas as pl
from jax.experimental.pallas import tpu as pltpu


def _gcn_kernel(adj_ref, inputx_ref, wr_ref, wu_ref, wp_ref, w1_ref,
                br_ref, bu_ref, bp_ref, b1_ref, w2_ref,
                nums_ref, b2_ref, wy_ref, by_ref, wa_ref, ba_ref,
                h_out, lsm_out, out_out, aa_out,
                xsel_s, wcat_s, c_s, z_s):
    p = pl.program_id(0)
    i = pl.program_id(1)
    n_rows = xsel_s.shape[0]
    bm = adj_ref.shape[0]

    @pl.when((p == 0) & (i == 0))
    def _build():
        wcat_s[...] = jnp.zeros(wcat_s.shape, jnp.float32)
        wcat_s[0:5, :] = wr_ref[...]
        wcat_s[7:8, :] = br_ref[...]
        wcat_s[8:15, :] = wu_ref[...]
        wcat_s[15:16, :] = bu_ref[...]
        wcat_s[16:22, :] = wp_ref[...]
        wcat_s[23:24, :] = bp_ref[...]
        c_s[...] = jnp.dot(wcat_s[...], w1_ref[...],
                           precision=jax.lax.Precision.HIGHEST,
                           preferred_element_type=jnp.float32)
        xin = jnp.concatenate(
            [inputx_ref[...], jnp.ones((n_rows, 1), jnp.float32)], axis=1)
        idx = jax.lax.broadcasted_iota(jnp.int32, (n_rows, 1), 0)
        n00 = nums_ref[0, 0]
        n01 = nums_ref[0, 1]
        n10 = nums_ref[1, 0]
        n11 = nums_ref[1, 1]
        n20 = nums_ref[2, 0]
        n21 = nums_ref[2, 1]
        seg2 = n10 != n11
        seg3 = n20 != n21
        mask_r = ((idx < n00)
                  | (seg2 & (idx >= n01) & (idx < n10))
                  | (seg3 & (idx >= n11) & (idx < n20)))
        mask_u = (((idx >= n00) & (idx < n01))
                  | (seg2 & (idx >= n10) & (idx < n11))
                  | (seg3 & (idx >= n20) & (idx < n21)))
        mask_p = idx >= n21
        xsel_s[...] = jnp.concatenate(
            [xin * mask_r.astype(jnp.float32),
             xin * mask_u.astype(jnp.float32),
             xin * mask_p.astype(jnp.float32)], axis=1)

    @pl.when(p == 0)
    def _phase0():
        u = jnp.dot(adj_ref[...], xsel_s[...],
                    precision=jax.lax.Precision.HIGHEST,
                    preferred_element_type=jnp.float32)
        h = jnp.maximum(
            jnp.dot(u, c_s[...], precision=jax.lax.Precision.HIGHEST,
                    preferred_element_type=jnp.float32)
            + b1_ref[...],
            0.0)
        z = jnp.dot(h, w2_ref[...], precision=jax.lax.Precision.HIGHEST,
                    preferred_element_type=jnp.float32)
        z_s[pl.ds(i * bm, bm), :] = z

    @pl.when(p == 1)
    def _phase1():
        h2 = jnp.dot(adj_ref[...], z_s[...],
                      precision=jax.lax.Precision.HIGHEST,
                      preferred_element_type=jnp.float32)
        c0 = h2[:, 0:1] + b2_ref[0, 0]
        c1 = h2[:, 1:2] + b2_ref[0, 1]
        h_out[:, 0:1] = c0
        h_out[:, 1:2] = c1
        m = jnp.maximum(c0, c1)
        lse = m + jnp.log(jnp.exp(c0 - m) + jnp.exp(c1 - m))
        lsm_out[:, 0:1] = c0 - lse
        lsm_out[:, 1:2] = c1 - lse
        y0 = c0 * wy_ref[0, 0] + c1 * wy_ref[1, 0] + by_ref[0, 0]
        y1 = c0 * wy_ref[0, 1] + c1 * wy_ref[1, 1] + by_ref[0, 1]
        out_out[:, 0:1] = jax.nn.sigmoid(y0)
        out_out[:, 1:2] = jax.nn.sigmoid(y1)
        a0 = c0 * wa_ref[0, 0] + c1 * wa_ref[1, 0] + ba_ref[0, 0]
        a1 = c0 * wa_ref[0, 1] + c1 * wa_ref[1, 1] + ba_ref[0, 1]
        aa_out[:, 0:1] = jax.nn.sigmoid(a0)
        aa_out[:, 1:2] = jax.nn.sigmoid(a1)


def kernel(inputx, adj, nums, Wr, br, Wu, bu, Wp, bp, W1, b1, W2, b2,
           Wy, by, Wa, ba):
    n = adj.shape[0]
    f = W1.shape[0]
    bm = 512
    nblk = n // bm

    vspec_whole = lambda shape: pl.BlockSpec(
        shape, lambda p, i: tuple(0 for _ in shape))
    smem = pl.BlockSpec(memory_space=pltpu.SMEM)
    out_spec = pl.BlockSpec((bm, 2), lambda p, i: (p * i, 0))

    outs = pl.pallas_call(
        _gcn_kernel,
        grid=(2, nblk),
        in_specs=[
            pl.BlockSpec((bm, n), lambda p, i: (i, 0)),
            vspec_whole((n, 7)),
            vspec_whole((5, f)),
            vspec_whole((7, f)),
            vspec_whole((6, f)),
            vspec_whole((f, f)),
            vspec_whole((1, f)),
            vspec_whole((1, f)),
            vspec_whole((1, f)),
            vspec_whole((1, f)),
            vspec_whole((f, 2)),
            smem,
            smem,
            smem,
            smem,
            smem,
            smem,
        ],
        out_specs=[out_spec, out_spec, out_spec, out_spec],
        out_shape=[jax.ShapeDtypeStruct((n, 2), jnp.float32)] * 4,
        scratch_shapes=[
            pltpu.VMEM((n, 24), jnp.float32),
            pltpu.VMEM((24, f), jnp.float32),
            pltpu.VMEM((24, f), jnp.float32),
            pltpu.VMEM((n, 2), jnp.float32),
        ],
        compiler_params=pltpu.CompilerParams(
            dimension_semantics=("arbitrary", "arbitrary")),
    )(adj, inputx, Wr, Wu, Wp, W1,
      br.reshape(1, f), bu.reshape(1, f), bp.reshape(1, f), b1.reshape(1, f),
      W2, nums, b2.reshape(1, 2), Wy, by.reshape(1, 2), Wa, ba.reshape(1, 2))
    h, lsm, out, aa = outs
    return (h, lsm, out, aa)


# speedup vs baseline: 1.8041x; 1.8041x over previous
import jax
import jax.numpy as jnp
from jax.experimental import pallas as pl
from jax.experimental.pallas import tpu as pltpu


def _split_hi_lo(a):
    hi = a.astype(jnp.bfloat16)
    lo = (a - hi.astype(jnp.float32)).astype(jnp.bfloat16)
    return hi, lo


def _gcn_kernel(adj_ref, inputx_ref, wr_ref, wu_ref, wp_ref, w1_ref,
                br_ref, bu_ref, bp_ref, b1_ref, w2_ref,
                nums_ref, b2_ref, wy_ref, by_ref, wa_ref, ba_ref,
                h_out, lsm_out, out_out, aa_out,
                xhl_s, wcat_s, c_s, zhl_s):
    p = pl.program_id(0)
    i = pl.program_id(1)
    n_rows = xhl_s.shape[0]
    bm = adj_ref.shape[0]

    @pl.when((p == 0) & (i == 0))
    def _build():
        wcat_s[...] = jnp.zeros(wcat_s.shape, jnp.float32)
        wcat_s[0:5, :] = wr_ref[...]
        wcat_s[7:8, :] = br_ref[...]
        wcat_s[8:15, :] = wu_ref[...]
        wcat_s[15:16, :] = bu_ref[...]
        wcat_s[16:22, :] = wp_ref[...]
        wcat_s[23:24, :] = bp_ref[...]
        c_s[...] = jnp.dot(wcat_s[...], w1_ref[...],
                           precision=jax.lax.Precision.HIGHEST,
                           preferred_element_type=jnp.float32)
        xin = jnp.concatenate(
            [inputx_ref[...], jnp.ones((n_rows, 1), jnp.float32)], axis=1)
        idx = jax.lax.broadcasted_iota(jnp.int32, (n_rows, 1), 0)
        n00 = nums_ref[0, 0]
        n01 = nums_ref[0, 1]
        n10 = nums_ref[1, 0]
        n11 = nums_ref[1, 1]
        n20 = nums_ref[2, 0]
        n21 = nums_ref[2, 1]
        seg2 = n10 != n11
        seg3 = n20 != n21
        mask_r = ((idx < n00)
                  | (seg2 & (idx >= n01) & (idx < n10))
                  | (seg3 & (idx >= n11) & (idx < n20)))
        mask_u = (((idx >= n00) & (idx < n01))
                  | (seg2 & (idx >= n10) & (idx < n11))
                  | (seg3 & (idx >= n20) & (idx < n21)))
        mask_p = idx >= n21
        xsel = jnp.concatenate(
            [xin * mask_r.astype(jnp.float32),
             xin * mask_u.astype(jnp.float32),
             xin * mask_p.astype(jnp.float32)], axis=1)
        xhi, xlo = _split_hi_lo(xsel)
        xhl_s[:, 0:24] = xhi
        xhl_s[:, 24:48] = xlo

    @pl.when(p == 0)
    def _phase0():
        adjh, adjl = _split_hi_lo(adj_ref[...])
        d1 = jnp.dot(adjh, xhl_s[...], preferred_element_type=jnp.float32)
        d2 = jnp.dot(adjl, xhl_s[:, 0:24],
                     preferred_element_type=jnp.float32)
        u = d1[:, 0:24] + d1[:, 24:48] + d2
        h = jnp.maximum(
            jnp.dot(u, c_s[...], precision=jax.lax.Precision.HIGHEST,
                    preferred_element_type=jnp.float32)
            + b1_ref[...],
            0.0)
        z = jnp.dot(h, w2_ref[...], precision=jax.lax.Precision.HIGHEST,
                    preferred_element_type=jnp.float32)
        zhi, zlo = _split_hi_lo(z)
        zhl_s[pl.ds(i * bm, bm), 0:2] = zhi
        zhl_s[pl.ds(i * bm, bm), 2:4] = zlo

    @pl.when(p == 1)
    def _phase1():
        adjh, adjl = _split_hi_lo(adj_ref[...])
        e1 = jnp.dot(adjh, zhl_s[...], preferred_element_type=jnp.float32)
        e2 = jnp.dot(adjl, zhl_s[:, 0:2],
                     preferred_element_type=jnp.float32)
        h2 = e1[:, 0:2] + e1[:, 2:4] + e2
        c0 = h2[:, 0:1] + b2_ref[0, 0]
        c1 = h2[:, 1:2] + b2_ref[0, 1]
        h_out[:, 0:1] = c0
        h_out[:, 1:2] = c1
        m = jnp.maximum(c0, c1)
        lse = m + jnp.log(jnp.exp(c0 - m) + jnp.exp(c1 - m))
        lsm_out[:, 0:1] = c0 - lse
        lsm_out[:, 1:2] = c1 - lse
        y0 = c0 * wy_ref[0, 0] + c1 * wy_ref[1, 0] + by_ref[0, 0]
        y1 = c0 * wy_ref[0, 1] + c1 * wy_ref[1, 1] + by_ref[0, 1]
        out_out[:, 0:1] = jax.nn.sigmoid(y0)
        out_out[:, 1:2] = jax.nn.sigmoid(y1)
        a0 = c0 * wa_ref[0, 0] + c1 * wa_ref[1, 0] + ba_ref[0, 0]
        a1 = c0 * wa_ref[0, 1] + c1 * wa_ref[1, 1] + ba_ref[0, 1]
        aa_out[:, 0:1] = jax.nn.sigmoid(a0)
        aa_out[:, 1:2] = jax.nn.sigmoid(a1)


def kernel(inputx, adj, nums, Wr, br, Wu, bu, Wp, bp, W1, b1, W2, b2,
           Wy, by, Wa, ba):
    n = adj.shape[0]
    f = W1.shape[0]
    bm = 512
    nblk = n // bm

    vspec_whole = lambda shape: pl.BlockSpec(
        shape, lambda p, i: tuple(0 for _ in shape))
    smem = pl.BlockSpec(memory_space=pltpu.SMEM)
    out_spec = pl.BlockSpec((bm, 2), lambda p, i: (p * i, 0))

    outs = pl.pallas_call(
        _gcn_kernel,
        grid=(2, nblk),
        in_specs=[
            pl.BlockSpec((bm, n), lambda p, i: (i, 0)),
            vspec_whole((n, 7)),
            vspec_whole((5, f)),
            vspec_whole((7, f)),
            vspec_whole((6, f)),
            vspec_whole((f, f)),
            vspec_whole((1, f)),
            vspec_whole((1, f)),
            vspec_whole((1, f)),
            vspec_whole((1, f)),
            vspec_whole((f, 2)),
            smem,
            smem,
            smem,
            smem,
            smem,
            smem,
        ],
        out_specs=[out_spec, out_spec, out_spec, out_spec],
        out_shape=[jax.ShapeDtypeStruct((n, 2), jnp.float32)] * 4,
        scratch_shapes=[
            pltpu.VMEM((n, 48), jnp.bfloat16),
            pltpu.VMEM((24, f), jnp.float32),
            pltpu.VMEM((24, f), jnp.float32),
            pltpu.VMEM((n, 4), jnp.bfloat16),
        ],
        compiler_params=pltpu.CompilerParams(
            dimension_semantics=("arbitrary", "arbitrary")),
    )(adj, inputx, Wr, Wu, Wp, W1,
      br.reshape(1, f), bu.reshape(1, f), bp.reshape(1, f), b1.reshape(1, f),
      W2, nums, b2.reshape(1, 2), Wy, by.reshape(1, 2), Wa, ba.reshape(1, 2))
    h, lsm, out, aa = outs
    return (h, lsm, out, aa)


# speedup vs baseline: 2.3238x; 1.2881x over previous
import jax
import jax.numpy as jnp
from jax.experimental import pallas as pl
from jax.experimental.pallas import tpu as pltpu


def _gcn_kernel(adj_ref, inputx_ref, wr_ref, wu_ref, wp_ref, w1_ref,
                br_ref, bu_ref, bp_ref, b1_ref, w2_ref,
                nums_ref, b2_ref, wy_ref, by_ref, wa_ref, ba_ref,
                h_out, lsm_out, out_out, aa_out,
                t_s, z_s):
    p = pl.program_id(0)
    i = pl.program_id(1)
    n_rows = t_s.shape[0]
    bm = adj_ref.shape[0]

    @pl.when((p == 0) & (i == 0))
    def _build():
        n00 = nums_ref[0, 0]
        n01 = nums_ref[0, 1]
        n10 = nums_ref[1, 0]
        n11 = nums_ref[1, 1]
        n20 = nums_ref[2, 0]
        n21 = nums_ref[2, 1]
        seg2 = n10 != n11
        seg3 = n20 != n21
        nc = 4
        cm = n_rows // nc
        for c in range(nc):
            r0 = c * cm
            xc = inputx_ref[r0:r0 + cm, :]
            xr = jnp.dot(xc[:, 0:5], wr_ref[...],
                         preferred_element_type=jnp.float32) + br_ref[...]
            xu = jnp.dot(xc[:, 0:7], wu_ref[...],
                         preferred_element_type=jnp.float32) + bu_ref[...]
            xp = jnp.dot(xc[:, 0:6], wp_ref[...],
                         preferred_element_type=jnp.float32) + bp_ref[...]
            idx = r0 + jax.lax.broadcasted_iota(jnp.int32, (cm, 1), 0)
            mask_r = ((idx < n00)
                      | (seg2 & (idx >= n01) & (idx < n10))
                      | (seg3 & (idx >= n11) & (idx < n20)))
            mask_u = (((idx >= n00) & (idx < n01))
                      | (seg2 & (idx >= n10) & (idx < n11))
                      | (seg3 & (idx >= n20) & (idx < n21)))
            mask_p = idx >= n21
            x = jnp.where(mask_r, xr,
                          jnp.where(mask_u, xu,
                                    jnp.where(mask_p, xp, 0.0)))
            t_s[r0:r0 + cm, :] = jnp.dot(
                x, w1_ref[...], preferred_element_type=jnp.float32)

    @pl.when(p == 0)
    def _phase0():
        h = jnp.maximum(
            jnp.dot(adj_ref[...], t_s[...],
                    preferred_element_type=jnp.float32) + b1_ref[...],
            0.0)
        z_s[pl.ds(i * bm, bm), :] = jnp.dot(
            h, w2_ref[...], preferred_element_type=jnp.float32)

    @pl.when(p == 1)
    def _phase1():
        h2 = jnp.dot(adj_ref[...], z_s[...],
                     preferred_element_type=jnp.float32)
        c0 = h2[:, 0:1] + b2_ref[0, 0]
        c1 = h2[:, 1:2] + b2_ref[0, 1]
        h_out[:, 0:1] = c0
        h_out[:, 1:2] = c1
        m = jnp.maximum(c0, c1)
        lse = m + jnp.log(jnp.exp(c0 - m) + jnp.exp(c1 - m))
        lsm_out[:, 0:1] = c0 - lse
        lsm_out[:, 1:2] = c1 - lse
        y0 = c0 * wy_ref[0, 0] + c1 * wy_ref[1, 0] + by_ref[0, 0]
        y1 = c0 * wy_ref[0, 1] + c1 * wy_ref[1, 1] + by_ref[0, 1]
        out_out[:, 0:1] = jax.nn.sigmoid(y0)
        out_out[:, 1:2] = jax.nn.sigmoid(y1)
        a0 = c0 * wa_ref[0, 0] + c1 * wa_ref[1, 0] + ba_ref[0, 0]
        a1 = c0 * wa_ref[0, 1] + c1 * wa_ref[1, 1] + ba_ref[0, 1]
        aa_out[:, 0:1] = jax.nn.sigmoid(a0)
        aa_out[:, 1:2] = jax.nn.sigmoid(a1)


def kernel(inputx, adj, nums, Wr, br, Wu, bu, Wp, bp, W1, b1, W2, b2,
           Wy, by, Wa, ba):
    n = adj.shape[0]
    f = W1.shape[0]
    bm = 512
    nblk = n // bm

    vspec_whole = lambda shape: pl.BlockSpec(
        shape, lambda p, i: tuple(0 for _ in shape))
    smem = pl.BlockSpec(memory_space=pltpu.SMEM)
    out_spec = pl.BlockSpec((bm, 2), lambda p, i: (p * i, 0))

    outs = pl.pallas_call(
        _gcn_kernel,
        grid=(2, nblk),
        in_specs=[
            pl.BlockSpec((bm, n), lambda p, i: (i, 0)),
            vspec_whole((n, 7)),
            vspec_whole((5, f)),
            vspec_whole((7, f)),
            vspec_whole((6, f)),
            vspec_whole((f, f)),
            vspec_whole((1, f)),
            vspec_whole((1, f)),
            vspec_whole((1, f)),
            vspec_whole((1, f)),
            vspec_whole((f, 2)),
            smem,
            smem,
            smem,
            smem,
            smem,
            smem,
        ],
        out_specs=[out_spec, out_spec, out_spec, out_spec],
        out_shape=[jax.ShapeDtypeStruct((n, 2), jnp.float32)] * 4,
        scratch_shapes=[
            pltpu.VMEM((n, f), jnp.float32),
            pltpu.VMEM((n, 2), jnp.float32),
        ],
        compiler_params=pltpu.CompilerParams(
            dimension_semantics=("arbitrary", "arbitrary")),
    )(adj, inputx, Wr, Wu, Wp, W1,
      br.reshape(1, f), bu.reshape(1, f), bp.reshape(1, f), b1.reshape(1, f),
      W2, nums, b2.reshape(1, 2), Wy, by.reshape(1, 2), Wa, ba.reshape(1, 2))
    h, lsm, out, aa = outs
    return (h, lsm, out, aa)


# speedup vs baseline: 2.4190x; 1.0410x over previous
import jax
import jax.numpy as jnp
from jax.experimental import pallas as pl
from jax.experimental.pallas import tpu as pltpu


def _gcn_kernel(adj_ref, inputx_ref, wr_ref, wu_ref, wp_ref, w1_ref,
                br_ref, bu_ref, bp_ref, b1_ref, w2_ref,
                nums_ref, b2_ref, wy_ref, by_ref, wa_ref, ba_ref,
                h_out, lsm_out, out_out, aa_out,
                t_s, z_s):
    p = pl.program_id(0)
    i = pl.program_id(1)
    n_rows = t_s.shape[0]
    bm = adj_ref.shape[0]

    @pl.when((p == 0) & (i == 0))
    def _build():
        n00 = nums_ref[0, 0]
        n01 = nums_ref[0, 1]
        n10 = nums_ref[1, 0]
        n11 = nums_ref[1, 1]
        n20 = nums_ref[2, 0]
        n21 = nums_ref[2, 1]
        seg2 = n10 != n11
        seg3 = n20 != n21
        nc = 4
        cm = n_rows // nc
        for c in range(nc):
            r0 = c * cm
            xc = inputx_ref[r0:r0 + cm, :]
            xr = jnp.dot(xc[:, 0:5], wr_ref[...],
                         preferred_element_type=jnp.float32) + br_ref[...]
            xu = jnp.dot(xc[:, 0:7], wu_ref[...],
                         preferred_element_type=jnp.float32) + bu_ref[...]
            xp = jnp.dot(xc[:, 0:6], wp_ref[...],
                         preferred_element_type=jnp.float32) + bp_ref[...]
            idx = r0 + jax.lax.broadcasted_iota(jnp.int32, (cm, 1), 0)
            mask_r = ((idx < n00)
                      | (seg2 & (idx >= n01) & (idx < n10))
                      | (seg3 & (idx >= n11) & (idx < n20)))
            mask_u = (((idx >= n00) & (idx < n01))
                      | (seg2 & (idx >= n10) & (idx < n11))
                      | (seg3 & (idx >= n20) & (idx < n21)))
            mask_p = idx >= n21
            x = jnp.where(mask_r, xr,
                          jnp.where(mask_u, xu,
                                    jnp.where(mask_p, xp, 0.0)))
            t_s[r0:r0 + cm, :] = jnp.dot(
                x, w1_ref[...], preferred_element_type=jnp.float32)

    @pl.when(p == 0)
    def _phase0():
        h = jnp.maximum(
            jnp.dot(adj_ref[...], t_s[...],
                    preferred_element_type=jnp.float32) + b1_ref[...],
            0.0)
        z_s[pl.ds(i * bm, bm), :] = jnp.dot(
            h, w2_ref[...], preferred_element_type=jnp.float32)

    @pl.when(p == 1)
    def _phase1():
        h2 = jnp.dot(adj_ref[...], z_s[...],
                     preferred_element_type=jnp.float32)
        c0 = h2[:, 0:1] + b2_ref[0, 0]
        c1 = h2[:, 1:2] + b2_ref[0, 1]
        h_out[:, 0:1] = c0
        h_out[:, 1:2] = c1
        m = jnp.maximum(c0, c1)
        lse = m + jnp.log(jnp.exp(c0 - m) + jnp.exp(c1 - m))
        lsm_out[:, 0:1] = c0 - lse
        lsm_out[:, 1:2] = c1 - lse
        y0 = c0 * wy_ref[0, 0] + c1 * wy_ref[1, 0] + by_ref[0, 0]
        y1 = c0 * wy_ref[0, 1] + c1 * wy_ref[1, 1] + by_ref[0, 1]
        out_out[:, 0:1] = jax.nn.sigmoid(y0)
        out_out[:, 1:2] = jax.nn.sigmoid(y1)
        a0 = c0 * wa_ref[0, 0] + c1 * wa_ref[1, 0] + ba_ref[0, 0]
        a1 = c0 * wa_ref[0, 1] + c1 * wa_ref[1, 1] + ba_ref[0, 1]
        aa_out[:, 0:1] = jax.nn.sigmoid(a0)
        aa_out[:, 1:2] = jax.nn.sigmoid(a1)


def kernel(inputx, adj, nums, Wr, br, Wu, bu, Wp, bp, W1, b1, W2, b2,
           Wy, by, Wa, ba):
    n = adj.shape[0]
    f = W1.shape[0]
    bm = 1024
    nblk = n // bm

    vspec_whole = lambda shape: pl.BlockSpec(
        shape, lambda p, i: tuple(0 for _ in shape))
    smem = pl.BlockSpec(memory_space=pltpu.SMEM)
    out_spec = pl.BlockSpec((bm, 2), lambda p, i: (p * i, 0))

    outs = pl.pallas_call(
        _gcn_kernel,
        grid=(2, nblk),
        in_specs=[
            pl.BlockSpec((bm, n), lambda p, i: (i, 0)),
            vspec_whole((n, 7)),
            vspec_whole((5, f)),
            vspec_whole((7, f)),
            vspec_whole((6, f)),
            vspec_whole((f, f)),
            vspec_whole((1, f)),
            vspec_whole((1, f)),
            vspec_whole((1, f)),
            vspec_whole((1, f)),
            vspec_whole((f, 2)),
            smem,
            smem,
            smem,
            smem,
            smem,
            smem,
        ],
        out_specs=[out_spec, out_spec, out_spec, out_spec],
        out_shape=[jax.ShapeDtypeStruct((n, 2), jnp.float32)] * 4,
        scratch_shapes=[
            pltpu.VMEM((n, f), jnp.float32),
            pltpu.VMEM((n, 2), jnp.float32),
        ],
        compiler_params=pltpu.CompilerParams(
            dimension_semantics=("arbitrary", "arbitrary")),
    )(adj, inputx, Wr, Wu, Wp, W1,
      br.reshape(1, f), bu.reshape(1, f), bp.reshape(1, f), b1.reshape(1, f),
      W2, nums, b2.reshape(1, 2), Wy, by.reshape(1, 2), Wa, ba.reshape(1, 2))
    h, lsm, out, aa = outs
    return (h, lsm, out, aa)
